# baseline (speedup 1.0000x reference)
import functools

import jax
import jax.numpy as jnp
from jax import lax
from jax.experimental import pallas as pl
from jax.experimental.pallas import tpu as pltpu
from jax.experimental.pallas import tpu_sc as plsc

_N = 10000
_E = 320000
_DIN = 128
_DH = 64

_NC = 2
_NS = 16
_NW = _NC * _NS
_C = 128
_NCH = 80
_EPAD = _NW * _NCH * _C
_NP = 10240
_RPS = _NP // _NS

_MESH = dict(core_axis_name="c", subcore_axis_name="s", num_cores=_NC,
             num_subcores=_NS)


@functools.partial(
    pl.kernel,
    out_type=jax.ShapeDtypeStruct((_NC, _NP), jnp.float32),
    mesh=plsc.VectorSubcoreMesh(**_MESH),
    scratch_types=[
        pltpu.VMEM((_NCH, _C), jnp.int32),
        pltpu.VMEM((_C,), jnp.float32),
        [pltpu.SemaphoreType.DMA for _ in range(4)],
        pltpu.VMEM_SHARED((_NP,), jnp.float32),
    ],
    compiler_params=pltpu.CompilerParams(use_tc_tiling_on_sc=False),
)
def _deg(dst_hbm, ones_hbm, zeros_hbm, out_hbm, didx, ones_v, ssem, acc):
    c = lax.axis_index("c")
    s = lax.axis_index("s")
    wid = c * _NS + s
    pltpu.sync_copy(dst_hbm.at[wid], didx)
    pltpu.sync_copy(ones_hbm, ones_v)
    pltpu.sync_copy(zeros_hbm, acc.at[pl.ds(s * _RPS, _RPS)])
    plsc.subcore_barrier()

    def scat(j, b):
        pltpu.async_copy(ones_v, acc.at[didx.at[j]], ssem[b], add=True)

    def scat_wait(j, b):
        pltpu.make_async_copy(ones_v, acc.at[didx.at[j]], ssem[b]).wait()

    for j in range(4):
        scat(j, j)

    def body(i, carry):
        j0 = 4 + 4 * i
        for k in range(4):
            scat_wait(j0 + k - 4, k)
            scat(j0 + k, k)
        return carry

    lax.fori_loop(0, (_NCH - 4) // 4, body, 0)
    for j in range(_NCH - 4, _NCH):
        scat_wait(j, j % 4)
    plsc.subcore_barrier()
    pltpu.sync_copy(acc.at[pl.ds(s * _RPS, _RPS)],
                    out_hbm.at[c, pl.ds(s * _RPS, _RPS)])


@functools.partial(
    pl.kernel,
    out_type=jax.ShapeDtypeStruct((_NP, _NC * _DH), jnp.float32),
    mesh=plsc.VectorSubcoreMesh(**_MESH),
    scratch_types=[
        pltpu.VMEM((_NCH // 2, 2 * _C), jnp.int32),
        pltpu.VMEM((_NCH, _C), jnp.int32),
        [pltpu.VMEM((2 * _C, _DH), jnp.float32) for _ in range(4)],
        [pltpu.SemaphoreType.DMA for _ in range(4)],
        [pltpu.SemaphoreType.DMA for _ in range(4)],
        pltpu.VMEM_SHARED((_NP, _DH), jnp.float32),
    ],
    compiler_params=pltpu.CompilerParams(use_tc_tiling_on_sc=False),
)
def _agg(g_hbm, src_hbm, dst_hbm, zeros_hbm, out_hbm,
         sidx, didx, rows, gsem, ssem, acc):
    c = lax.axis_index("c")
    s = lax.axis_index("s")
    wid = c * _NS + s
    pltpu.sync_copy(src_hbm.at[wid], sidx)
    pltpu.sync_copy(dst_hbm.at[wid], didx)
    pltpu.sync_copy(zeros_hbm, acc.at[pl.ds(s * _RPS, _RPS)])
    plsc.subcore_barrier()

    NJ = _NCH // 2

    def gath(J, b):
        pltpu.async_copy(g_hbm.at[sidx.at[J]], rows[b], gsem[b])

    def gath_wait(J, b):
        pltpu.make_async_copy(g_hbm.at[sidx.at[J]], rows[b], gsem[b]).wait()

    def scat2(J, b):
        lo = rows[b].at[pl.ds(0, _C)]
        hi = rows[b].at[pl.ds(_C, _C)]
        pltpu.async_copy(lo, acc.at[didx.at[2 * J]], ssem[b], add=True)
        pltpu.async_copy(hi, acc.at[didx.at[2 * J + 1]], ssem[b], add=True)

    def scat2_wait(J, b):
        lo = rows[b].at[pl.ds(0, _C)]
        hi = rows[b].at[pl.ds(_C, _C)]
        pltpu.make_async_copy(lo, acc.at[didx.at[2 * J]], ssem[b]).wait()
        pltpu.make_async_copy(hi, acc.at[didx.at[2 * J + 1]], ssem[b]).wait()

    gath(0, 0)
    gath(1, 1)
    for J in range(2):
        gath_wait(J, J)
        scat2(J, J)
        gath(J + 2, J + 2)

    def body(i, carry):
        J0 = 2 + 4 * i
        for k in range(4):
            J = J0 + k
            b = (2 + k) % 4
            gath_wait(J, b)
            scat2(J, b)
            bn = (b + 2) % 4
            scat2_wait(J - 2, bn)
            gath(J + 2, bn)
        return carry

    lax.fori_loop(0, (NJ - 8) // 4, body, 0)
    for J in range(NJ - 6, NJ):
        b = J % 4
        gath_wait(J, b)
        scat2(J, b)
        bn = (b + 2) % 4
        scat2_wait(J - 2, bn)
        if J + 2 < NJ:
            gath(J + 2, bn)
    for J in range(NJ - 2, NJ):
        scat2_wait(J, J % 4)
    plsc.subcore_barrier()
    pltpu.sync_copy(acc.at[pl.ds(s * _RPS, _RPS)],
                    out_hbm.at[pl.ds(s * _RPS, _RPS), pl.ds(c * _DH, _DH)])


_BLK = 1024
_G = _NP // _BLK


def _dis_block(degp_ref):
    deg = degp_ref[0:1, :] + degp_ref[1:2, :] + 1.0
    return jnp.transpose(lax.rsqrt(deg), (1, 0))


def _mm1_body(x_ref, w_ref, degp_ref, out_ref):
    dis = _dis_block(degp_ref)
    h = jnp.dot(x_ref[...], w_ref[...], preferred_element_type=jnp.float32)
    out_ref[...] = h * dis


def _mm2_body(sp_ref, g_ref, degp_ref, b_ref, w_ref, out_ref):
    dis = _dis_block(degp_ref)
    ssum = sp_ref[:, :_DH] + sp_ref[:, _DH:] + g_ref[...]
    h = jnp.maximum(ssum * dis + b_ref[...], 0.0)
    out_ref[...] = jnp.dot(h, w_ref[...],
                           preferred_element_type=jnp.float32) * dis


def _mm3_body(sp_ref, g_ref, degp_ref, b_ref, wd_ref, bd_ref,
              h_ref, dec_ref):
    dis = _dis_block(degp_ref)
    ssum = sp_ref[:, :_DH] + sp_ref[:, _DH:] + g_ref[...]
    h = jnp.maximum(ssum * dis + b_ref[...], 0.0)
    h_ref[...] = h
    dec_ref[...] = jnp.dot(h, wd_ref[...],
                           preferred_element_type=jnp.float32) + bd_ref[...]


def _mm1(x, W1, degp):
    return pl.pallas_call(
        _mm1_body,
        grid=(_G,),
        in_specs=[
            pl.BlockSpec((_BLK, _DIN), lambda i: (i, 0)),
            pl.BlockSpec((_DIN, _DH), lambda i: (0, 0)),
            pl.BlockSpec((_NC, _BLK), lambda i: (0, i)),
        ],
        out_specs=pl.BlockSpec((_BLK, _DH), lambda i: (i, 0)),
        out_shape=jax.ShapeDtypeStruct((_NP, _DH), jnp.float32),
    )(x, W1, degp)


def _mm2(sp, g1, degp, b1, W2):
    return pl.pallas_call(
        _mm2_body,
        grid=(_G,),
        in_specs=[
            pl.BlockSpec((_BLK, _NC * _DH), lambda i: (i, 0)),
            pl.BlockSpec((_BLK, _DH), lambda i: (i, 0)),
            pl.BlockSpec((_NC, _BLK), lambda i: (0, i)),
            pl.BlockSpec((1, _DH), lambda i: (0, 0)),
            pl.BlockSpec((_DH, _DH), lambda i: (0, 0)),
        ],
        out_specs=pl.BlockSpec((_BLK, _DH), lambda i: (i, 0)),
        out_shape=jax.ShapeDtypeStruct((_NP, _DH), jnp.float32),
    )(sp, g1, degp, b1, W2)


def _mm3(sp, g2, degp, b2, Wd, bd):
    return pl.pallas_call(
        _mm3_body,
        grid=(_G,),
        in_specs=[
            pl.BlockSpec((_BLK, _NC * _DH), lambda i: (i, 0)),
            pl.BlockSpec((_BLK, _DH), lambda i: (i, 0)),
            pl.BlockSpec((_NC, _BLK), lambda i: (0, i)),
            pl.BlockSpec((1, _DH), lambda i: (0, 0)),
            pl.BlockSpec((_DH, _DIN), lambda i: (0, 0)),
            pl.BlockSpec((1, _DIN), lambda i: (0, 0)),
        ],
        out_specs=[
            pl.BlockSpec((_BLK, _DH), lambda i: (i, 0)),
            pl.BlockSpec((_BLK, _DIN), lambda i: (i, 0)),
        ],
        out_shape=[
            jax.ShapeDtypeStruct((_NP, _DH), jnp.float32),
            jax.ShapeDtypeStruct((_NP, _DIN), jnp.float32),
        ],
    )(sp, g2, degp, b2, Wd, bd)


def kernel(x, edge_index, W1, b1, W2, b2, Wd, bd):
    ei = edge_index.astype(jnp.int32)
    pad = _N + jnp.arange(_EPAD - _E, dtype=jnp.int32) % (_NP - _N)
    src = jnp.concatenate([ei[0], pad]).reshape(_NW, _NCH // 2, 2 * _C)
    dst = jnp.concatenate([ei[1], pad]).reshape(_NW, _NCH, _C)

    ones1 = jnp.ones((_C,), jnp.float32)
    zeros1 = jnp.zeros((_RPS,), jnp.float32)
    zeros64 = jnp.zeros((_RPS, _DH), jnp.float32)
    b1r = b1.reshape(1, _DH)
    b2r = b2.reshape(1, _DH)
    bdr = bd.reshape(1, _DIN)

    degp = _deg(dst, ones1, zeros1)
    g1 = _mm1(x, W1, degp)
    sp1 = _agg(g1, src, dst, zeros64)
    g2 = _mm2(sp1, g1, degp, b1r, W2)
    sp2 = _agg(g2, src, dst, zeros64)
    h, dec = _mm3(sp2, g2, degp, b2r, Wd, bdr)
    return h[:_N], dec[:_N]


# speedup vs baseline: 1.0028x; 1.0028x over previous
import functools

import jax
import jax.numpy as jnp
from jax import lax
from jax.experimental import pallas as pl
from jax.experimental.pallas import tpu as pltpu
from jax.experimental.pallas import tpu_sc as plsc

_N = 10000
_E = 320000
_DIN = 128
_DH = 64

_NC = 2
_NS = 16
_NW = _NC * _NS
_C = 128
_NCH = 80
_EPAD = _NW * _NCH * _C
_NP = 10240
_RPS = _NP // _NS

_MESH = dict(core_axis_name="c", subcore_axis_name="s", num_cores=_NC,
             num_subcores=_NS)


@functools.partial(
    pl.kernel,
    out_type=jax.ShapeDtypeStruct((_NC, _NP), jnp.float32),
    mesh=plsc.VectorSubcoreMesh(**_MESH),
    scratch_types=[
        pltpu.VMEM((_NCH, _C), jnp.int32),
        pltpu.VMEM((_C,), jnp.float32),
        [pltpu.SemaphoreType.DMA for _ in range(4)],
        pltpu.VMEM_SHARED((_NP,), jnp.float32),
    ],
    compiler_params=pltpu.CompilerParams(use_tc_tiling_on_sc=False),
)
def _deg(dst_hbm, ones_hbm, zeros_hbm, out_hbm, didx, ones_v, ssem, acc):
    c = lax.axis_index("c")
    s = lax.axis_index("s")
    wid = c * _NS + s
    pltpu.sync_copy(dst_hbm.at[wid], didx)
    pltpu.sync_copy(ones_hbm, ones_v)
    pltpu.sync_copy(zeros_hbm, acc.at[pl.ds(s * _RPS, _RPS)])
    plsc.subcore_barrier()

    def scat(j, b):
        pltpu.async_copy(ones_v, acc.at[didx.at[j]], ssem[b], add=True)

    def scat_wait(j, b):
        pltpu.make_async_copy(ones_v, acc.at[didx.at[j]], ssem[b]).wait()

    for j in range(4):
        scat(j, j)

    def body(i, carry):
        j0 = 4 + 4 * i
        for k in range(4):
            scat_wait(j0 + k - 4, k)
            scat(j0 + k, k)
        return carry

    lax.fori_loop(0, (_NCH - 4) // 4, body, 0)
    for j in range(_NCH - 4, _NCH):
        scat_wait(j, j % 4)
    plsc.subcore_barrier()
    pltpu.sync_copy(acc.at[pl.ds(s * _RPS, _RPS)],
                    out_hbm.at[c, pl.ds(s * _RPS, _RPS)])


@functools.partial(
    pl.kernel,
    out_type=jax.ShapeDtypeStruct((_NP, _NC * _DH), jnp.float32),
    mesh=plsc.VectorSubcoreMesh(**_MESH),
    scratch_types=[
        pltpu.VMEM((_NCH, _C), jnp.int32),
        pltpu.VMEM((_NCH, _C), jnp.int32),
        [pltpu.VMEM((_C, _DH), jnp.float32) for _ in range(6)],
        [pltpu.SemaphoreType.DMA for _ in range(6)],
        [pltpu.SemaphoreType.DMA for _ in range(6)],
        pltpu.VMEM_SHARED((_NP, _DH), jnp.float32),
    ],
    compiler_params=pltpu.CompilerParams(use_tc_tiling_on_sc=False),
)
def _agg(g_hbm, src_hbm, dst_hbm, zeros_hbm, out_hbm,
         sidx, didx, rows, gsem, ssem, acc):
    c = lax.axis_index("c")
    s = lax.axis_index("s")
    wid = c * _NS + s
    pltpu.sync_copy(src_hbm.at[wid], sidx)
    pltpu.sync_copy(dst_hbm.at[wid], didx)
    pltpu.sync_copy(zeros_hbm, acc.at[pl.ds(s * _RPS, _RPS)])
    plsc.subcore_barrier()

    def gath(j, b):
        pltpu.async_copy(g_hbm.at[sidx.at[j]], rows[b], gsem[b])

    def gath_wait(j, b):
        pltpu.make_async_copy(g_hbm.at[sidx.at[j]], rows[b], gsem[b]).wait()

    def scat(j, b):
        pltpu.async_copy(rows[b], acc.at[didx.at[j]], ssem[b], add=True)

    def scat_wait(j, b):
        pltpu.make_async_copy(rows[b], acc.at[didx.at[j]], ssem[b]).wait()

    for j in range(3):
        gath(j, j)
    for j in range(3):
        gath_wait(j, j)
        scat(j, j)
        gath(j + 3, j + 3)

    def body(i, carry):
        j0 = 3 + 6 * i
        for k in range(6):
            j = j0 + k
            b = (3 + k) % 6
            gath_wait(j, b)
            scat(j, b)
            bn = (b + 3) % 6
            scat_wait(j - 3, bn)
            gath(j + 3, bn)
        return carry

    lax.fori_loop(0, (_NCH - 8) // 6, body, 0)
    for j in range(_NCH - 5, _NCH):
        b = j % 6
        gath_wait(j, b)
        scat(j, b)
        bn = (b + 3) % 6
        scat_wait(j - 3, bn)
        if j + 3 < _NCH:
            gath(j + 3, bn)
    for j in range(_NCH - 3, _NCH):
        scat_wait(j, j % 6)
    plsc.subcore_barrier()
    pltpu.sync_copy(acc.at[pl.ds(s * _RPS, _RPS)],
                    out_hbm.at[pl.ds(s * _RPS, _RPS), pl.ds(c * _DH, _DH)])


_BLK = 1024
_G = _NP // _BLK


def _dis_block(degp_ref):
    deg = degp_ref[0:1, :] + degp_ref[1:2, :] + 1.0
    return jnp.transpose(lax.rsqrt(deg), (1, 0))


def _mm1_body(x_ref, w_ref, degp_ref, out_ref):
    dis = _dis_block(degp_ref)
    h = jnp.dot(x_ref[...], w_ref[...], preferred_element_type=jnp.float32)
    out_ref[...] = h * dis


def _mm2_body(sp_ref, g_ref, degp_ref, b_ref, w_ref, out_ref):
    dis = _dis_block(degp_ref)
    ssum = sp_ref[:, :_DH] + sp_ref[:, _DH:] + g_ref[...]
    h = jnp.maximum(ssum * dis + b_ref[...], 0.0)
    out_ref[...] = jnp.dot(h, w_ref[...],
                           preferred_element_type=jnp.float32) * dis


def _mm3_body(sp_ref, g_ref, degp_ref, b_ref, wd_ref, bd_ref,
              h_ref, dec_ref):
    dis = _dis_block(degp_ref)
    ssum = sp_ref[:, :_DH] + sp_ref[:, _DH:] + g_ref[...]
    h = jnp.maximum(ssum * dis + b_ref[...], 0.0)
    h_ref[...] = h
    dec_ref[...] = jnp.dot(h, wd_ref[...],
                           preferred_element_type=jnp.float32) + bd_ref[...]


def _mm1(x, W1, degp):
    return pl.pallas_call(
        _mm1_body,
        grid=(_G,),
        in_specs=[
            pl.BlockSpec((_BLK, _DIN), lambda i: (i, 0)),
            pl.BlockSpec((_DIN, _DH), lambda i: (0, 0)),
            pl.BlockSpec((_NC, _BLK), lambda i: (0, i)),
        ],
        out_specs=pl.BlockSpec((_BLK, _DH), lambda i: (i, 0)),
        out_shape=jax.ShapeDtypeStruct((_NP, _DH), jnp.float32),
    )(x, W1, degp)


def _mm2(sp, g1, degp, b1, W2):
    return pl.pallas_call(
        _mm2_body,
        grid=(_G,),
        in_specs=[
            pl.BlockSpec((_BLK, _NC * _DH), lambda i: (i, 0)),
            pl.BlockSpec((_BLK, _DH), lambda i: (i, 0)),
            pl.BlockSpec((_NC, _BLK), lambda i: (0, i)),
            pl.BlockSpec((1, _DH), lambda i: (0, 0)),
            pl.BlockSpec((_DH, _DH), lambda i: (0, 0)),
        ],
        out_specs=pl.BlockSpec((_BLK, _DH), lambda i: (i, 0)),
        out_shape=jax.ShapeDtypeStruct((_NP, _DH), jnp.float32),
    )(sp, g1, degp, b1, W2)


def _mm3(sp, g2, degp, b2, Wd, bd):
    return pl.pallas_call(
        _mm3_body,
        grid=(_G,),
        in_specs=[
            pl.BlockSpec((_BLK, _NC * _DH), lambda i: (i, 0)),
            pl.BlockSpec((_BLK, _DH), lambda i: (i, 0)),
            pl.BlockSpec((_NC, _BLK), lambda i: (0, i)),
            pl.BlockSpec((1, _DH), lambda i: (0, 0)),
            pl.BlockSpec((_DH, _DIN), lambda i: (0, 0)),
            pl.BlockSpec((1, _DIN), lambda i: (0, 0)),
        ],
        out_specs=[
            pl.BlockSpec((_BLK, _DH), lambda i: (i, 0)),
            pl.BlockSpec((_BLK, _DIN), lambda i: (i, 0)),
        ],
        out_shape=[
            jax.ShapeDtypeStruct((_NP, _DH), jnp.float32),
            jax.ShapeDtypeStruct((_NP, _DIN), jnp.float32),
        ],
    )(sp, g2, degp, b2, Wd, bd)


def kernel(x, edge_index, W1, b1, W2, b2, Wd, bd):
    ei = edge_index.astype(jnp.int32)
    pad = _N + jnp.arange(_EPAD - _E, dtype=jnp.int32) % (_NP - _N)
    src = jnp.concatenate([ei[0], pad]).reshape(_NW, _NCH, _C)
    dst = jnp.concatenate([ei[1], pad]).reshape(_NW, _NCH, _C)

    ones1 = jnp.ones((_C,), jnp.float32)
    zeros1 = jnp.zeros((_RPS,), jnp.float32)
    zeros64 = jnp.zeros((_RPS, _DH), jnp.float32)
    b1r = b1.reshape(1, _DH)
    b2r = b2.reshape(1, _DH)
    bdr = bd.reshape(1, _DIN)

    degp = _deg(dst, ones1, zeros1)
    g1 = _mm1(x, W1, degp)
    sp1 = _agg(g1, src, dst, zeros64)
    g2 = _mm2(sp1, g1, degp, b1r, W2)
    sp2 = _agg(g2, src, dst, zeros64)
    h, dec = _mm3(sp2, g2, degp, b2r, Wd, bdr)
    return h[:_N], dec[:_N]
